# Initial kernel scaffold; baseline (speedup 1.0000x reference)
import jax
import jax.numpy as jnp
from jax.experimental import pallas as pl


def kernel(inputs, W_ih, W_hh, b_ih, b_hh, W_out, b_out, W_root, W_pool, b_pool):
    raise NotImplementedError("write your pallas kernel here")


# speedup vs baseline: 10.1796x; 10.1796x over previous
import jax
import jax.numpy as jnp
from jax.experimental import pallas as pl
from jax.experimental.pallas import tpu as pltpu

N, T, D, H, KNN, C = 2048, 32, 128, 128, 16, 16
BLK = 256
NBLK = N // BLK

_DN = (((1,), (1,)), ((), ()))


def _lstm_body(x_ref, wih_ref, whh_ref, bih_ref, bhh_ref, h_ref):
    wih = wih_ref[...]
    whh = whh_ref[...]
    bih = bih_ref[...]
    bhh = bhh_ref[...]

    def step(t, carry):
        h, c = carry
        x_t = x_ref[t]
        gates = (
            jax.lax.dot_general(x_t, wih, _DN, preferred_element_type=jnp.float32)
            + bih
            + jax.lax.dot_general(h.astype(jnp.bfloat16), whh, _DN,
                                  preferred_element_type=jnp.float32)
            + bhh
        )
        i = jax.nn.sigmoid(gates[:, 0 * H : 1 * H])
        f = jax.nn.sigmoid(gates[:, 1 * H : 2 * H])
        g = jnp.tanh(gates[:, 2 * H : 3 * H])
        o = jax.nn.sigmoid(gates[:, 3 * H : 4 * H])
        c = f * c + i * g
        h = o * jnp.tanh(c)
        return (h, c)

    z = jnp.zeros((BLK, H), jnp.float32)
    h, _ = jax.lax.fori_loop(0, T, step, (z, z))
    h_ref[...] = h


def _graph_body(xb_ref, xall_ref, wout_ref, wroot_ref, bout_ref, wpool_ref,
                bpool_ref, m_ref, s_ref):
    pid = pl.program_id(0)
    xb = xb_ref[...]
    xall = xall_ref[...]

    g = jax.lax.dot_general(xb, xall, _DN, preferred_element_type=jnp.float32)
    sq_b = jnp.sum(xb * xb, axis=1, keepdims=True)
    sq_a = jnp.sum(xall * xall, axis=1, keepdims=True).T
    rows = jax.lax.broadcasted_iota(jnp.int32, (BLK, N), 0) + pid * BLK
    cols = jax.lax.broadcasted_iota(jnp.int32, (BLK, N), 1)
    d2 = (sq_b + sq_a) - 2.0 * g
    d2 = d2 + jnp.where(rows == cols, 1e12, 0.0)
    vals = -d2

    neg = -jnp.inf
    for _ in range(KNN):
        mx = jnp.max(vals, axis=1, keepdims=True)
        cand = jnp.where(vals == mx, cols, jnp.int32(2 * N))
        amin = jnp.min(cand, axis=1, keepdims=True)
        vals = jnp.where(cols == amin, neg, vals)
    mask = jnp.where(vals == neg, 1.0, 0.0)

    xn = jnp.dot(mask, xall, preferred_element_type=jnp.float32)
    agg = (xn + xb) * (1.0 / (KNN + 1))
    h2 = (
        (jnp.dot(agg, wout_ref[...], preferred_element_type=jnp.float32)
         + bout_ref[...])
        + jnp.dot(xb, wroot_ref[...], preferred_element_type=jnp.float32)
    )
    x2 = jnp.maximum(h2, 0.0).astype(jnp.bfloat16).astype(jnp.float32)

    z = jnp.dot(x2, wpool_ref[...], preferred_element_type=jnp.float32) + bpool_ref[...]
    z = z - jnp.max(z, axis=1, keepdims=True)
    e = jnp.exp(z)
    s = e / jnp.sum(e, axis=1, keepdims=True)

    m_ref[...] = mask
    s_ref[...] = s


def _pool_body(m_ref, sall_ref, spec_ref, orth_ref, clus_ref,
               adj_acc, ss_acc, cs_acc, deg_acc):
    pid = pl.program_id(0)

    @pl.when(pid == 0)
    def _():
        adj_acc[...] = jnp.zeros_like(adj_acc)
        ss_acc[...] = jnp.zeros_like(ss_acc)
        cs_acc[...] = jnp.zeros_like(cs_acc)
        deg_acc[...] = jnp.zeros_like(deg_acc)

    sall = sall_ref[...]
    sblk = sall_ref[pl.ds(pid * BLK, BLK), :]
    mask = m_ref[...]

    p = jnp.dot(mask, sall, preferred_element_type=jnp.float32)
    p = p.astype(jnp.bfloat16).astype(jnp.float32)

    adj_acc[...] += jax.lax.dot_general(
        p, sblk, (((0,), (0,)), ((), ())), preferred_element_type=jnp.float32
    )
    ss_acc[...] += jax.lax.dot_general(
        sblk, sblk, (((0,), (0,)), ((), ())), preferred_element_type=jnp.float32
    )
    cs_acc[...] += jnp.sum(sblk, axis=0, keepdims=True)
    deg_acc[...] += jnp.sum(mask, axis=0, keepdims=True)

    @pl.when(pid == NBLK - 1)
    def _():
        m_edges = jnp.float32(N * KNN / 2.0)
        out_adj = adj_acc[...]
        ca = jnp.dot(deg_acc[...], sall, preferred_element_type=jnp.float32)
        eye = jnp.where(
            jax.lax.broadcasted_iota(jnp.int32, (C, C), 0)
            == jax.lax.broadcasted_iota(jnp.int32, (C, C), 1),
            1.0,
            0.0,
        )
        norm_diag = (ca * ca) / 2.0 / m_edges
        diag = jnp.sum(out_adj * eye, axis=1)[None]
        tr = jnp.sum(diag - norm_diag)
        spec_ref[...] = jnp.full((1, 1), -tr / 2.0 / m_edges)

        ss = ss_acc[...]
        ss_n = jnp.sqrt(jnp.sum(ss * ss))
        ortho = ss / ss_n - eye / jnp.sqrt(jnp.float32(C))
        orth_ref[...] = jnp.full((1, 1), jnp.sqrt(jnp.sum(ortho * ortho)))

        cs = cs_acc[...]
        clus_ref[...] = jnp.full(
            (1, 1),
            jnp.sqrt(jnp.sum(cs * cs)) / N * jnp.sqrt(jnp.float32(C)) - 1.0,
        )


def kernel(inputs, W_ih, W_hh, b_ih, b_hh, W_out, b_out, W_root, W_pool, b_pool):
    x = pl.pallas_call(
        _lstm_body,
        grid=(NBLK,),
        in_specs=[
            pl.BlockSpec((T, BLK, D), lambda i: (0, i, 0)),
            pl.BlockSpec((4 * H, D), lambda i: (0, 0)),
            pl.BlockSpec((4 * H, H), lambda i: (0, 0)),
            pl.BlockSpec((1, 4 * H), lambda i: (0, 0)),
            pl.BlockSpec((1, 4 * H), lambda i: (0, 0)),
        ],
        out_specs=pl.BlockSpec((BLK, H), lambda i: (i, 0)),
        out_shape=jax.ShapeDtypeStruct((N, H), jnp.float32),
    )(
        jnp.swapaxes(inputs, 0, 1).astype(jnp.bfloat16),
        W_ih.astype(jnp.bfloat16),
        W_hh.astype(jnp.bfloat16),
        b_ih.reshape(1, 4 * H),
        b_hh.reshape(1, 4 * H),
    )

    mask, s = pl.pallas_call(
        _graph_body,
        grid=(NBLK,),
        in_specs=[
            pl.BlockSpec((BLK, H), lambda i: (i, 0)),
            pl.BlockSpec((N, H), lambda i: (0, 0)),
            pl.BlockSpec((H, H), lambda i: (0, 0)),
            pl.BlockSpec((H, H), lambda i: (0, 0)),
            pl.BlockSpec((1, H), lambda i: (0, 0)),
            pl.BlockSpec((H, C), lambda i: (0, 0)),
            pl.BlockSpec((1, C), lambda i: (0, 0)),
        ],
        out_specs=[
            pl.BlockSpec((BLK, N), lambda i: (i, 0)),
            pl.BlockSpec((BLK, C), lambda i: (i, 0)),
        ],
        out_shape=[
            jax.ShapeDtypeStruct((N, N), jnp.float32),
            jax.ShapeDtypeStruct((N, C), jnp.float32),
        ],
    )(x, x, W_out, W_root, b_out.reshape(1, H), W_pool, b_pool.reshape(1, C))

    spec, orth, clus = pl.pallas_call(
        _pool_body,
        grid=(NBLK,),
        in_specs=[
            pl.BlockSpec((BLK, N), lambda i: (i, 0)),
            pl.BlockSpec((N, C), lambda i: (0, 0)),
        ],
        out_specs=[
            pl.BlockSpec((1, 1), lambda i: (0, 0)),
            pl.BlockSpec((1, 1), lambda i: (0, 0)),
            pl.BlockSpec((1, 1), lambda i: (0, 0)),
        ],
        out_shape=[
            jax.ShapeDtypeStruct((1, 1), jnp.float32),
            jax.ShapeDtypeStruct((1, 1), jnp.float32),
            jax.ShapeDtypeStruct((1, 1), jnp.float32),
        ],
        scratch_shapes=[
            pltpu.VMEM((C, C), jnp.float32),
            pltpu.VMEM((C, C), jnp.float32),
            pltpu.VMEM((1, C), jnp.float32),
            pltpu.VMEM((1, N), jnp.float32),
        ],
    )(mask, s)

    return s[None], spec[0, 0], orth[0, 0], clus[0, 0]
